# Initial kernel scaffold; baseline (speedup 1.0000x reference)
import functools
import numpy as np
import jax
import jax.numpy as jnp
from jax import lax
from jax.experimental import pallas as pl
from jax.experimental.pallas import tpu as pltpu
from jax.experimental.pallas import tpu_sc as plsc

EPS = 1e-4
B0 = 136
B1 = 68
T0SIZE = 2424832
T1SIZE = 309760

NC, NS = 2, 16
NW = NC * NS
IB = 128

_mesh = plsc.VectorSubcoreMesh(core_axis_name="c", subcore_axis_name="s")


def _wid():
    return lax.axis_index("s") * NC + lax.axis_index("c")


def _k1_body(chunk, pos3, t0, t1, poskey, k1key, px, py, pz, k0v, k1v, valv):
    r = chunk // IB
    wid = _wid()
    base = wid * chunk
    pltpu.sync_copy(pos3.at[0, pl.ds(base, chunk)], px)
    pltpu.sync_copy(pos3.at[1, pl.ds(base, chunk)], py)
    pltpu.sync_copy(pos3.at[2, pl.ds(base, chunk)], pz)

    @pl.loop(0, r)
    def _(j):
        @pl.loop(0, IB, step=16)
        def _(i):
            f = pl.ds(j * IB + i, 16)
            s = pl.ds(i, 16)
            x, y, z = px[f], py[f], pz[f]
            k0v[j, s] = ((x + 2) * B0 + (y + 2)) * B0 + (z + 2)
            x1, y1, z1 = x >> 1, y >> 1, z >> 1
            k1v[j, s] = ((x1 + 2) * B1 + (y1 + 2)) * B1 + (z1 + 2)
            valv[j, s] = lax.iota(jnp.int32, 16) + (base + j * IB + i)

        pltpu.sync_copy(k0v.at[j], poskey.at[pl.ds(base + j * IB, IB)])
        pltpu.sync_copy(k1v.at[j], k1key.at[pl.ds(base + j * IB, IB)])
        pltpu.sync_copy(valv.at[j], t0.at[k0v.at[j]])
        pltpu.sync_copy(valv.at[j], t1.at[k1v.at[j]])


def _k2_body(chunk, n_real, pos3, t0, t1, poskey, k1key,
             nbr0, dg, nbr1, kidx,
             px, py, pz, k0v, kev, k1v, qkv, cv, ccv, vkv, nbv):
    r = chunk // IB
    wid = _wid()
    base = wid * chunk
    pltpu.sync_copy(pos3.at[0, pl.ds(base, chunk)], px)
    pltpu.sync_copy(pos3.at[1, pl.ds(base, chunk)], py)
    pltpu.sync_copy(pos3.at[2, pl.ds(base, chunk)], pz)

    @pl.loop(0, r)
    def _(j):
        @pl.loop(0, IB, step=16)
        def _(i):
            f = pl.ds(j * IB + i, 16)
            s = pl.ds(i, 16)
            x, y, z = px[f], py[f], pz[f]
            k0v[j, s] = ((x + 2) * B0 + (y + 2)) * B0 + (z + 2)
            ex, ey, ez = x & -2, y & -2, z & -2
            kev[j, s] = ((ex + 2) * B0 + (ey + 2)) * B0 + (ez + 2)
            x1, y1, z1 = x >> 1, y >> 1, z >> 1
            k1v[j, s] = ((x1 + 2) * B1 + (y1 + 2)) * B1 + (z1 + 2)
            nbv[j, s] = (x & 1) * 4 + (y & 1) * 2 + (z & 1)

        pltpu.sync_copy(nbv.at[j], kidx.at[pl.ds(base + j * IB, IB)])

    def lookup(keyv, dk, tab, karr, out_ref, k):
        @pl.loop(0, r)
        def _(j):
            @pl.loop(0, IB, step=16)
            def _(i):
                s = pl.ds(i, 16)
                qkv[j, s] = keyv[j, s] + dk

            pltpu.sync_copy(tab.at[qkv.at[j]], cv.at[j])

            @pl.loop(0, IB, step=16)
            def _(i):
                s = pl.ds(i, 16)
                c = cv[j, s]
                ccv[j, s] = jnp.minimum(jnp.maximum(c, 0), n_real - 1)

            pltpu.sync_copy(karr.at[ccv.at[j]], vkv.at[j])

            @pl.loop(0, IB, step=16)
            def _(i):
                s = pl.ds(i, 16)
                c = cv[j, s]
                ok = (c >= 0) & (c < n_real) & (vkv[j, s] == qkv[j, s])
                nbv[j, s] = jnp.where(ok, ccv[j, s], n_real)

            pltpu.sync_copy(nbv.at[j], out_ref.at[k, pl.ds(base + j * IB, IB)])

    @pl.loop(0, 27)
    def _(k):
        di, dj, dl = k // 9 - 1, (k // 3) % 3 - 1, k % 3 - 1
        lookup(k0v, (di * B0 + dj) * B0 + dl, t0, poskey, nbr0, k)

    @pl.loop(0, 8)
    def _(k):
        di, dj, dl = k // 4, (k // 2) % 2, k % 2
        lookup(kev, (di * B0 + dj) * B0 + dl, t0, poskey, dg, k)

    @pl.loop(0, 27)
    def _(k):
        di, dj, dl = k // 9 - 1, (k // 3) % 3 - 1, k % 3 - 1
        lookup(k1v, (di * B1 + dj) * B1 + dl, t1, k1key, nbr1, k)


def _k3_body(chunk, nk, c_dim, idxs, feat, gout, idxv, rows):
    r = chunk // IB
    wid = _wid()
    base = wid * chunk
    for k in range(nk):
        @pl.loop(0, r)
        def _(j):
            pltpu.sync_copy(idxs.at[k, pl.ds(base + j * IB, IB)], idxv.at[j])
            pltpu.sync_copy(feat.at[idxv.at[j]],
                            rows.at[pl.ds(j * IB, IB), :])

        pltpu.sync_copy(rows, gout.at[pl.ds(base, chunk),
                                      pl.ds(k * c_dim, c_dim)])


def _sc_call(body, out_types, scratch, *args):
    fn = pl.kernel(body, out_type=out_types, mesh=_mesh,
                   scratch_types=scratch)
    return fn(*args)


def _bn_kernel(n_real, br, x_ref, s_ref, b_ref, o_ref):
    pid = pl.program_id(0)
    y = jnp.maximum(x_ref[...] * s_ref[...] + b_ref[...], 0.0)
    rows = jax.lax.broadcasted_iota(jnp.int32, y.shape, 0) + pid * br
    o_ref[...] = jnp.where(rows < n_real, y, 0.0)


def _bn_relu_pad(x, g, b, n_real, br=512):
    npad, c = x.shape
    s = (g / np.sqrt(1.0 + EPS)).reshape(1, c)
    b = b.reshape(1, c)
    return pl.pallas_call(
        functools.partial(_bn_kernel, n_real, br),
        grid=(npad // br,),
        in_specs=[pl.BlockSpec((br, c), lambda i: (i, 0)),
                  pl.BlockSpec((1, c), lambda i: (0, 0)),
                  pl.BlockSpec((1, c), lambda i: (0, 0))],
        out_specs=pl.BlockSpec((br, c), lambda i: (i, 0)),
        out_shape=jax.ShapeDtypeStruct((npad, c), jnp.float32),
    )(x, s, b)


def _mm_kernel(n_real, br, bn, g_ref, w_ref, s_ref, b_ref, o_ref):
    pid = pl.program_id(0)
    acc = jnp.dot(g_ref[...], w_ref[...], preferred_element_type=jnp.float32)
    if bn:
        acc = jnp.maximum(acc * s_ref[...] + b_ref[...], 0.0)
    rows = jax.lax.broadcasted_iota(jnp.int32, acc.shape, 0) + pid * br
    o_ref[...] = jnp.where(rows < n_real, acc, 0.0)


def _matmul(gmat, wf, gvec, bvec, n_real, bn, br=512):
    npad, kc = gmat.shape
    d = wf.shape[1]
    if gvec is None:
        gvec = jnp.ones((d,), jnp.float32)
        bvec = jnp.zeros((d,), jnp.float32)
    s = (gvec / np.sqrt(1.0 + EPS)).reshape(1, d)
    b = bvec.reshape(1, d)
    return pl.pallas_call(
        functools.partial(_mm_kernel, n_real, br, bn),
        grid=(npad // br,),
        in_specs=[pl.BlockSpec((br, kc), lambda i: (i, 0)),
                  pl.BlockSpec((kc, d), lambda i: (0, 0)),
                  pl.BlockSpec((1, d), lambda i: (0, 0)),
                  pl.BlockSpec((1, d), lambda i: (0, 0))],
        out_specs=pl.BlockSpec((br, d), lambda i: (i, 0)),
        out_shape=jax.ShapeDtypeStruct((npad, d), jnp.float32),
    )(gmat, wf, s, b)


def _up_kernel(n_real, br, u_ref, wu_ref, kx_ref, skip_ref, s_ref, b_ref,
               o_ref):
    pid = pl.program_id(0)
    p = jnp.dot(u_ref[...], wu_ref[...], preferred_element_type=jnp.float32)
    kx = kx_ref[...]
    up = jnp.zeros((br, 32), jnp.float32)
    for k in range(8):
        sel = (kx == k).astype(jnp.float32)
        up = up + sel * p[:, k * 32:(k + 1) * 32]
    cat = jnp.concatenate([skip_ref[...], up], axis=1)
    y = jnp.maximum(cat * s_ref[...] + b_ref[...], 0.0)
    rows = jax.lax.broadcasted_iota(jnp.int32, y.shape, 0) + pid * br
    o_ref[...] = jnp.where(rows < n_real, y, 0.0)


def _up_concat(u, wuf, kidx, skip, g5, b5, n_real, br=512):
    npad = u.shape[0]
    s = (g5 / np.sqrt(1.0 + EPS)).reshape(1, 64)
    b = b5.reshape(1, 64)
    return pl.pallas_call(
        functools.partial(_up_kernel, n_real, br),
        grid=(npad // br,),
        in_specs=[pl.BlockSpec((br, 64), lambda i: (i, 0)),
                  pl.BlockSpec((64, 256), lambda i: (0, 0)),
                  pl.BlockSpec((br, 1), lambda i: (i, 0)),
                  pl.BlockSpec((br, 32), lambda i: (i, 0)),
                  pl.BlockSpec((1, 64), lambda i: (0, 0)),
                  pl.BlockSpec((1, 64), lambda i: (0, 0))],
        out_specs=pl.BlockSpec((br, 64), lambda i: (i, 0)),
        out_shape=jax.ShapeDtypeStruct((npad, 64), jnp.float32),
    )(u, wuf, kidx, skip, s, b)


def kernel(feat, pos, W1, Wd, W2, Wu, W3,
           g1, b1, g2, b2, g3, b3, g4, b4, g5, b5):
    n = feat.shape[0]
    chunk = ((n + 1 + NW - 1) // NW + IB - 1) // IB * IB
    npad = NW * chunk

    i32 = jnp.int32
    pos3 = jnp.concatenate(
        [pos.astype(i32), jnp.full((npad - n, 3), -2, i32)], 0).T
    featp = jnp.concatenate(
        [feat, jnp.zeros((npad - n, 32), feat.dtype)], 0)

    sd = jax.ShapeDtypeStruct
    vm = pltpu.VMEM
    r = chunk // IB

    t0, t1, poskey, k1key = _sc_call(
        functools.partial(_k1_body, chunk),
        (sd((T0SIZE,), i32), sd((T1SIZE,), i32),
         sd((npad,), i32), sd((npad,), i32)),
        [vm((chunk,), i32)] * 3 + [vm((r, IB), i32)] * 3,
        pos3)

    nbr0, dg, nbr1, kidx = _sc_call(
        functools.partial(_k2_body, chunk, n),
        (sd((27, npad), i32), sd((8, npad), i32),
         sd((27, npad), i32), sd((npad,), i32)),
        [vm((chunk,), i32)] * 3 + [vm((r, IB), i32)] * 8,
        pos3, t0, t1, poskey, k1key)

    def gather(idxs, f, nk, c):
        return _sc_call(
            functools.partial(_k3_body, chunk, nk, c),
            sd((npad, nk * c), jnp.float32),
            [vm((r, IB), i32), vm((chunk, c), jnp.float32)],
            idxs, f)

    w1f = W1.reshape(27 * 32, 32)
    wdf = Wd.reshape(8 * 32, 64)
    w2f = W2.reshape(27 * 64, 64)
    wuf = Wu.transpose(1, 0, 2).reshape(64, 8 * 32)
    w3f = W3.reshape(27 * 64, 32)

    x1p = _bn_relu_pad(featp, g1, b1, n)
    g1m = gather(nbr0, x1p, 27, 32)
    x_raw = _matmul(g1m, w1f, None, None, n, bn=False)
    x2p = _bn_relu_pad(x_raw, g2, b2, n)
    g2m = gather(dg, x2p, 8, 32)
    y1p = _matmul(g2m, wdf, g3, b3, n, bn=True)
    g3m = gather(nbr1, y1p, 27, 64)
    y2p = _matmul(g3m, w2f, g4, b4, n, bn=True)
    um = gather(nbr1[13:14], y2p, 1, 64)
    x3p = _up_concat(um, wuf, kidx.reshape(npad, 1), x_raw, g5, b5, n)
    g4m = gather(nbr0, x3p, 27, 64)
    out = _matmul(g4m, w3f, None, None, n, bn=False)
    return out[:n]


# speedup vs baseline: 22.8563x; 22.8563x over previous
import functools
import numpy as np
import jax
import jax.numpy as jnp
from jax import lax
from jax.experimental import pallas as pl
from jax.experimental.pallas import tpu as pltpu
from jax.experimental.pallas import tpu_sc as plsc

EPS = 1e-4
B0 = 136
B1 = 68
T0SIZE = 2424832
T1SIZE = 309760

NC, NS = 2, 16
NW = NC * NS
IB = 128

_mesh = plsc.VectorSubcoreMesh(core_axis_name="c", subcore_axis_name="s")


def _wid():
    return lax.axis_index("s") * NC + lax.axis_index("c")


def _k1_body(chunk, posx, posy, posz, t0, t1, poskey, k1key,
             px, py, pz, k0v, k1v, valv, sem):
    wid = _wid()
    base = wid * chunk
    pltpu.sync_copy(posx.at[pl.ds(base, chunk)], px)
    pltpu.sync_copy(posy.at[pl.ds(base, chunk)], py)
    pltpu.sync_copy(posz.at[pl.ds(base, chunk)], pz)

    @pl.loop(0, chunk, step=16)
    def _(i):
        s = pl.ds(i, 16)
        x, y, z = px[s], py[s], pz[s]
        k0v[s] = ((x + 2) * B0 + (y + 2)) * B0 + (z + 2)
        x1, y1, z1 = x >> 1, y >> 1, z >> 1
        k1v[s] = ((x1 + 2) * B1 + (y1 + 2)) * B1 + (z1 + 2)
        valv[s] = lax.iota(jnp.int32, 16) + (base + i)

    pltpu.sync_copy(k0v, poskey.at[pl.ds(base, chunk)])
    pltpu.sync_copy(k1v, k1key.at[pl.ds(base, chunk)])
    hs = []
    for j in range(chunk // IB):
        s = pl.ds(j * IB, IB)
        hs.append(pltpu.async_copy(valv.at[s], t0.at[k0v.at[s]], sem))
        hs.append(pltpu.async_copy(valv.at[s], t1.at[k1v.at[s]], sem))
    for h in hs:
        h.wait()


def _k2_body(chunk, n_real, npad, posx, posy, posz, t0, t1, poskey, k1key,
             nbr0, dg, nbr1, kidx,
             px, py, pz, k0v, kev, k1v, qkv, cv, ccv, vkv, nbv, sem):
    wid = _wid()
    base = wid * chunk
    pltpu.sync_copy(posx.at[pl.ds(base, chunk)], px)
    pltpu.sync_copy(posy.at[pl.ds(base, chunk)], py)
    pltpu.sync_copy(posz.at[pl.ds(base, chunk)], pz)

    @pl.loop(0, chunk, step=16)
    def _(i):
        s = pl.ds(i, 16)
        x, y, z = px[s], py[s], pz[s]
        k0v[s] = ((x + 2) * B0 + (y + 2)) * B0 + (z + 2)
        ex, ey, ez = x & -2, y & -2, z & -2
        kev[s] = ((ex + 2) * B0 + (ey + 2)) * B0 + (ez + 2)
        x1, y1, z1 = x >> 1, y >> 1, z >> 1
        k1v[s] = ((x1 + 2) * B1 + (y1 + 2)) * B1 + (z1 + 2)
        nbv[s] = (x & 1) * 4 + (y & 1) * 2 + (z & 1)

    pltpu.sync_copy(nbv, kidx.at[pl.ds(base, chunk)])

    def lookup(keyv, dk, tab, karr, out_ref, k, tsize):
        @pl.loop(0, chunk, step=16)
        def _(i):
            s = pl.ds(i, 16)
            q = keyv[s] + dk
            qkv[s] = jnp.minimum(jnp.maximum(q, 0), tsize - 1)

        hs = [pltpu.async_copy(tab.at[qkv.at[pl.ds(j * IB, IB)]],
                               cv.at[pl.ds(j * IB, IB)], sem)
              for j in range(chunk // IB)]
        for h in hs:
            h.wait()

        @pl.loop(0, chunk, step=16)
        def _(i):
            s = pl.ds(i, 16)
            ccv[s] = jnp.minimum(jnp.maximum(cv[s], 0), n_real - 1)

        hs = [pltpu.async_copy(karr.at[ccv.at[pl.ds(j * IB, IB)]],
                               vkv.at[pl.ds(j * IB, IB)], sem)
              for j in range(chunk // IB)]
        for h in hs:
            h.wait()

        @pl.loop(0, chunk, step=16)
        def _(i):
            s = pl.ds(i, 16)
            c = cv[s]
            ok = (c >= 0) & (c < n_real) & (vkv[s] == qkv[s])
            nbv[s] = jnp.where(ok, ccv[s], n_real)

        off = pl.multiple_of(k * npad + base, IB)
        pltpu.sync_copy(nbv, out_ref.at[pl.ds(off, chunk)])

    @pl.loop(0, 27)
    def _(k):
        di, dj, dl = k // 9 - 1, (k // 3) % 3 - 1, k % 3 - 1
        lookup(k0v, (di * B0 + dj) * B0 + dl, t0, poskey, nbr0, k, T0SIZE)

    @pl.loop(0, 8)
    def _(k):
        di, dj, dl = k // 4, (k // 2) % 2, k % 2
        lookup(kev, (di * B0 + dj) * B0 + dl, t0, poskey, dg, k, T0SIZE)

    @pl.loop(0, 27)
    def _(k):
        di, dj, dl = k // 9 - 1, (k // 3) % 3 - 1, k % 3 - 1
        lookup(k1v, (di * B1 + dj) * B1 + dl, t1, k1key, nbr1, k, T1SIZE)


def _k3_body(chunk, npad, nk, c_dim, idxs, feat, gout, idxv, rows, sem):
    wid = _wid()
    base = wid * chunk
    for k in range(nk):
        off = pl.multiple_of(k * npad + base, IB)
        pltpu.sync_copy(idxs.at[pl.ds(off, chunk)], idxv)
        hs = [pltpu.async_copy(feat.at[idxv.at[pl.ds(j * IB, IB)]],
                               rows.at[pl.ds(j * IB, IB), :], sem)
              for j in range(chunk // IB)]
        for h in hs:
            h.wait()
        pltpu.sync_copy(rows, gout.at[pl.ds(base, chunk),
                                      pl.ds(k * c_dim, c_dim)])


def _k4_body(chh, n_real, npad, nk, cap, nbrs, dsts, srcs, cnts,
             nbuf, dbuf, sbuf, cbuf):
    wid = _wid()
    spc = chh // 16
    for v in range(2):
        hw = wid * 2 + v
        hbase = hw * chh

        @pl.loop(0, nk)
        def _(k):
            off = pl.multiple_of(k * npad + hbase, 8)
            koff = pl.multiple_of(k * chh, 8)
            pltpu.sync_copy(nbrs.at[pl.ds(off, chh)],
                            nbuf.at[pl.ds(koff, chh)])

        @pl.loop(0, cap, step=16)
        def _(i):
            dbuf[pl.ds(i, 16)] = jnp.full((16,), chh, jnp.int32)
            sbuf[pl.ds(i, 16)] = jnp.full((16,), n_real, jnp.int32)

        def step(i, cnt):
            vec = nbuf[pl.ds(i * 16, 16)]
            k = i // spc
            slot = (i % spc) * 16
            m = (vec != n_real) & (cnt < cap - 16)
            dv = lax.iota(jnp.int32, 16) + slot
            sv = vec + k * npad
            plsc.store_compressed(dbuf.at[pl.ds(cnt, 16)], dv, mask=m)
            plsc.store_compressed(sbuf.at[pl.ds(cnt, 16)], sv, mask=m)
            return cnt + jnp.sum(m.astype(jnp.int32))

        cnt = lax.fori_loop(0, nk * spc, step, jnp.int32(0))
        cbuf[pl.ds(0, 16)] = jnp.broadcast_to(cnt, (16,))
        pltpu.sync_copy(dbuf, dsts.at[pl.ds(hw * cap, cap)])
        pltpu.sync_copy(sbuf, srcs.at[pl.ds(hw * cap, cap)])
        pltpu.sync_copy(cbuf, cnts.at[pl.ds(hw * 16, 16)])


def _k5_body(chh, c_dim, cap, dsts, srcs, cnts, z, out,
             dbuf, sbuf, cbuf, rows, acc, sem):
    wid = _wid()
    for v in range(2):
        hw = wid * 2 + v
        pltpu.sync_copy(dsts.at[pl.ds(hw * cap, cap)], dbuf.at[pl.ds(0, cap)])
        pltpu.sync_copy(srcs.at[pl.ds(hw * cap, cap)], sbuf)
        pltpu.sync_copy(cnts.at[pl.ds(hw * 16, 16)], cbuf)
        cnt = cbuf[pl.ds(0, 16)][0]

        @pl.loop(0, chh + 8)
        def _(rr):
            @pl.loop(0, c_dim, step=16)
            def _(cc):
                acc[rr, pl.ds(cc, 16)] = jnp.zeros((16,), jnp.float32)

        nb = (cnt + 255) // 256

        def batch(j, _):
            pltpu.async_copy(z.at[sbuf.at[pl.ds(j * 256, 256)]], rows,
                             sem).wait()

            def pstep(p, _):
                d = dbuf[pl.ds(j * 256 + p, 16)][0]
                for cc in range(c_dim // 16):
                    s = pl.ds(cc * 16, 16)
                    acc[d, s] = acc[d, s] + rows[p, s]
                return 0

            lax.fori_loop(0, 256, pstep, 0)
            return 0

        lax.fori_loop(0, nb, batch, 0)
        pltpu.sync_copy(acc.at[pl.ds(0, chh), :],
                        out.at[pl.ds(hw * chh, chh), :])


def _sc_call(body, out_types, scratch, *args, layout_passes=True):
    fn = pl.kernel(body, out_type=out_types, mesh=_mesh,
                   scratch_types=scratch,
                   compiler_params=pltpu.CompilerParams(
                       use_tc_tiling_on_sc=False,
                       needs_layout_passes=layout_passes))
    return fn(*args)


def _bn_kernel(n_real, br, x_ref, s_ref, b_ref, o_ref):
    pid = pl.program_id(0)
    y = jnp.maximum(x_ref[...] * s_ref[...] + b_ref[...], 0.0)
    rows = jax.lax.broadcasted_iota(jnp.int32, y.shape, 0) + pid * br
    o_ref[...] = jnp.where(rows < n_real, y, 0.0)


def _bn_relu_pad(x, g, b, n_real, br=512):
    npad, c = x.shape
    s = (g / np.sqrt(1.0 + EPS)).reshape(1, c)
    b = b.reshape(1, c)
    return pl.pallas_call(
        functools.partial(_bn_kernel, n_real, br),
        grid=(npad // br,),
        in_specs=[pl.BlockSpec((br, c), lambda i: (i, 0)),
                  pl.BlockSpec((1, c), lambda i: (0, 0)),
                  pl.BlockSpec((1, c), lambda i: (0, 0))],
        out_specs=pl.BlockSpec((br, c), lambda i: (i, 0)),
        out_shape=jax.ShapeDtypeStruct((npad, c), jnp.float32),
    )(x, s, b)


def _zmm_kernel(n_real, br, x_ref, w_ref, o_ref):
    i = pl.program_id(0)
    acc = jnp.dot(x_ref[...], w_ref[...], preferred_element_type=jnp.float32)
    rows = jax.lax.broadcasted_iota(jnp.int32, acc.shape, 0) + i * br
    o_ref[...] = jnp.where(rows < n_real, acc, 0.0)


def _zmm(xp, wf, nk, n_real, br=512):
    npad, cin = xp.shape
    cout = wf.shape[1]
    nb = npad // br
    return pl.pallas_call(
        functools.partial(_zmm_kernel, n_real, br),
        grid=(nb, nk),
        in_specs=[pl.BlockSpec((br, cin), lambda i, k: (i, 0)),
                  pl.BlockSpec((cin, cout), lambda i, k: (k, 0))],
        out_specs=pl.BlockSpec((br, cout), lambda i, k: (k * nb + i, 0)),
        out_shape=jax.ShapeDtypeStruct((nk * npad, cout), jnp.float32),
    )(xp, wf)


def _up_kernel(n_real, br, u_ref, wu_ref, kx_ref, skip_ref, s_ref, b_ref,
               o_ref):
    pid = pl.program_id(0)
    p = jnp.dot(u_ref[...], wu_ref[...], preferred_element_type=jnp.float32)
    kx = kx_ref[...]
    up = jnp.zeros((br, 32), jnp.float32)
    for k in range(8):
        sel = (kx == k).astype(jnp.float32)
        up = up + sel * p[:, k * 32:(k + 1) * 32]
    cat = jnp.concatenate([skip_ref[...], up], axis=1)
    y = jnp.maximum(cat * s_ref[...] + b_ref[...], 0.0)
    rows = jax.lax.broadcasted_iota(jnp.int32, y.shape, 0) + pid * br
    o_ref[...] = jnp.where(rows < n_real, y, 0.0)


def _up_concat(u, wuf, kidx, skip, g5, b5, n_real, br=512):
    npad = u.shape[0]
    s = (g5 / np.sqrt(1.0 + EPS)).reshape(1, 64)
    b = b5.reshape(1, 64)
    return pl.pallas_call(
        functools.partial(_up_kernel, n_real, br),
        grid=(npad // br,),
        in_specs=[pl.BlockSpec((br, 64), lambda i: (i, 0)),
                  pl.BlockSpec((64, 256), lambda i: (0, 0)),
                  pl.BlockSpec((br, 1), lambda i: (i, 0)),
                  pl.BlockSpec((br, 32), lambda i: (i, 0)),
                  pl.BlockSpec((1, 64), lambda i: (0, 0)),
                  pl.BlockSpec((1, 64), lambda i: (0, 0))],
        out_specs=pl.BlockSpec((br, 64), lambda i: (i, 0)),
        out_shape=jax.ShapeDtypeStruct((npad, 64), jnp.float32),
    )(u, wuf, kidx, skip, s, b)


def kernel(feat, pos, W1, Wd, W2, Wu, W3,
           g1, b1, g2, b2, g3, b3, g4, b4, g5, b5):
    n = feat.shape[0]
    chunk = ((n + 1 + NW - 1) // NW + IB - 1) // IB * IB
    npad = NW * chunk
    chh = chunk // 2
    nw2 = NW * 2

    i32 = jnp.int32
    posp = jnp.concatenate(
        [pos.astype(i32), jnp.full((npad - n, 3), -2, i32)], 0)
    posx = posp[:, 0] * 1
    posy = posp[:, 1] * 1
    posz = posp[:, 2] * 1
    featp = jnp.concatenate(
        [feat, jnp.zeros((npad - n, 32), feat.dtype)], 0)

    sd = jax.ShapeDtypeStruct
    vm = pltpu.VMEM

    t0, t1, poskey, k1key = _sc_call(
        functools.partial(_k1_body, chunk),
        (sd((T0SIZE,), i32), sd((T1SIZE,), i32),
         sd((npad,), i32), sd((npad,), i32)),
        [vm((chunk,), i32)] * 6 + [pltpu.SemaphoreType.DMA],
        posx, posy, posz)

    nbr0, dg, nbr1, kidx = _sc_call(
        functools.partial(_k2_body, chunk, n, npad),
        (sd((27 * npad,), i32), sd((8 * npad,), i32),
         sd((27 * npad,), i32), sd((npad,), i32)),
        [vm((chunk,), i32)] * 11 + [pltpu.SemaphoreType.DMA],
        posx, posy, posz, t0, t1, poskey, k1key)

    def compact(nbrs, nk, cap):
        return _sc_call(
            functools.partial(_k4_body, chh, n, npad, nk, cap),
            (sd((nw2 * cap,), i32), sd((nw2 * cap,), i32),
             sd((nw2 * 16,), i32)),
            [vm((nk * chh,), i32), vm((cap,), i32), vm((cap,), i32),
             vm((16,), i32)],
            nbrs, layout_passes=False), cap

    def sconv(pairs, z, c):
        (dsts, srcs, cnts), cap = pairs
        return _sc_call(
            functools.partial(_k5_body, chh, c, cap),
            sd((npad, c), jnp.float32),
            [vm((cap + 16,), i32), vm((cap,), i32), vm((16,), i32),
             vm((256, c), jnp.float32), vm((chh + 8, c), jnp.float32),
             pltpu.SemaphoreType.DMA],
            dsts, srcs, cnts, z)

    def gather(idxs, f, nk, c):
        return _sc_call(
            functools.partial(_k3_body, chunk, npad, nk, c),
            sd((npad, nk * c), jnp.float32),
            [vm((chunk,), i32), vm((chunk, c), jnp.float32),
             pltpu.SemaphoreType.DMA],
            idxs, f)

    w1f = W1.reshape(27 * 32, 32)
    wdf = Wd.reshape(8 * 32, 64)
    w2f = W2.reshape(27 * 64, 64)
    wuf = Wu.transpose(1, 0, 2).reshape(64, 8 * 32)
    w3f = W3.reshape(27 * 64, 32)

    pairs0 = compact(nbr0, 27, 4096)
    pairsd = compact(dg, 8, 2048)
    pairs1 = compact(nbr1, 27, 8192)

    x1p = _bn_relu_pad(featp, g1, b1, n)
    x_raw = sconv(pairs0, _zmm(x1p, w1f, 27, n), 32)
    x2p = _bn_relu_pad(x_raw, g2, b2, n)
    xd = sconv(pairsd, _zmm(x2p, wdf, 8, n), 64)
    y1p = _bn_relu_pad(xd, g3, b3, n)
    y_raw = sconv(pairs1, _zmm(y1p, w2f, 27, n), 64)
    y2p = _bn_relu_pad(y_raw, g4, b4, n)
    um = gather(nbr1[13 * npad:14 * npad], y2p, 1, 64)
    x3p = _up_concat(um, wuf, kidx.reshape(npad, 1), x_raw, g5, b5, n)
    out = sconv(pairs0, _zmm(x3p, w3f, 27, n), 32)
    return out[:n]


# speedup vs baseline: 23.1845x; 1.0144x over previous
import functools
import numpy as np
import jax
import jax.numpy as jnp
from jax import lax
from jax.experimental import pallas as pl
from jax.experimental.pallas import tpu as pltpu
from jax.experimental.pallas import tpu_sc as plsc

EPS = 1e-4
B0 = 136
B1 = 68
T0SIZE = 2424832
T1SIZE = 309760

NC, NS = 2, 16
NW = NC * NS
IB = 128

_mesh = plsc.VectorSubcoreMesh(core_axis_name="c", subcore_axis_name="s")


def _wid():
    return lax.axis_index("s") * NC + lax.axis_index("c")


def _k1_body(chunk, posx, posy, posz, t0, t1, poskey, k1key,
             px, py, pz, k0v, k1v, valv, sem):
    wid = _wid()
    base = wid * chunk
    pltpu.sync_copy(posx.at[pl.ds(base, chunk)], px)
    pltpu.sync_copy(posy.at[pl.ds(base, chunk)], py)
    pltpu.sync_copy(posz.at[pl.ds(base, chunk)], pz)

    @pl.loop(0, chunk, step=16)
    def _(i):
        s = pl.ds(i, 16)
        x, y, z = px[s], py[s], pz[s]
        k0v[s] = ((x + 2) * B0 + (y + 2)) * B0 + (z + 2)
        x1, y1, z1 = x >> 1, y >> 1, z >> 1
        k1v[s] = ((x1 + 2) * B1 + (y1 + 2)) * B1 + (z1 + 2)
        valv[s] = lax.iota(jnp.int32, 16) + (base + i)

    pltpu.sync_copy(k0v, poskey.at[pl.ds(base, chunk)])
    pltpu.sync_copy(k1v, k1key.at[pl.ds(base, chunk)])
    hs = []
    for j in range(chunk // IB):
        s = pl.ds(j * IB, IB)
        hs.append(pltpu.async_copy(valv.at[s], t0.at[k0v.at[s]], sem))
        hs.append(pltpu.async_copy(valv.at[s], t1.at[k1v.at[s]], sem))
    for h in hs:
        h.wait()


def _k2_body(chunk, n_real, npad, posx, posy, posz, t0, t1, poskey, k1key,
             nbr0, dg, nbr1, kidx,
             px, py, pz, k0v, kev, k1v, qkv, cv, ccv, vkv, nbv, sem):
    wid = _wid()
    base = wid * chunk
    pltpu.sync_copy(posx.at[pl.ds(base, chunk)], px)
    pltpu.sync_copy(posy.at[pl.ds(base, chunk)], py)
    pltpu.sync_copy(posz.at[pl.ds(base, chunk)], pz)

    @pl.loop(0, chunk, step=16)
    def _(i):
        s = pl.ds(i, 16)
        x, y, z = px[s], py[s], pz[s]
        k0v[s] = ((x + 2) * B0 + (y + 2)) * B0 + (z + 2)
        ex, ey, ez = x & -2, y & -2, z & -2
        kev[s] = ((ex + 2) * B0 + (ey + 2)) * B0 + (ez + 2)
        x1, y1, z1 = x >> 1, y >> 1, z >> 1
        k1v[s] = ((x1 + 2) * B1 + (y1 + 2)) * B1 + (z1 + 2)
        nbv[s] = (x & 1) * 4 + (y & 1) * 2 + (z & 1)

    pltpu.sync_copy(nbv, kidx.at[pl.ds(base, chunk)])

    def lookup(keyv, dk, tab, karr, out_ref, k, tsize):
        @pl.loop(0, chunk, step=16)
        def _(i):
            s = pl.ds(i, 16)
            q = keyv[s] + dk
            qkv[s] = jnp.minimum(jnp.maximum(q, 0), tsize - 1)

        hs = [pltpu.async_copy(tab.at[qkv.at[pl.ds(j * IB, IB)]],
                               cv.at[pl.ds(j * IB, IB)], sem)
              for j in range(chunk // IB)]
        for h in hs:
            h.wait()

        @pl.loop(0, chunk, step=16)
        def _(i):
            s = pl.ds(i, 16)
            ccv[s] = jnp.minimum(jnp.maximum(cv[s], 0), n_real - 1)

        hs = [pltpu.async_copy(karr.at[ccv.at[pl.ds(j * IB, IB)]],
                               vkv.at[pl.ds(j * IB, IB)], sem)
              for j in range(chunk // IB)]
        for h in hs:
            h.wait()

        @pl.loop(0, chunk, step=16)
        def _(i):
            s = pl.ds(i, 16)
            c = cv[s]
            ok = (c >= 0) & (c < n_real) & (vkv[s] == qkv[s])
            nbv[s] = jnp.where(ok, ccv[s], n_real)

        off = pl.multiple_of(k * npad + base, IB)
        pltpu.sync_copy(nbv, out_ref.at[pl.ds(off, chunk)])

    @pl.loop(0, 27)
    def _(k):
        di, dj, dl = k // 9 - 1, (k // 3) % 3 - 1, k % 3 - 1
        lookup(k0v, (di * B0 + dj) * B0 + dl, t0, poskey, nbr0, k, T0SIZE)

    @pl.loop(0, 8)
    def _(k):
        di, dj, dl = k // 4, (k // 2) % 2, k % 2
        lookup(kev, (di * B0 + dj) * B0 + dl, t0, poskey, dg, k, T0SIZE)

    @pl.loop(0, 27)
    def _(k):
        di, dj, dl = k // 9 - 1, (k // 3) % 3 - 1, k % 3 - 1
        lookup(k1v, (di * B1 + dj) * B1 + dl, t1, k1key, nbr1, k, T1SIZE)


def _k3_body(chunk, npad, nk, c_dim, idxs, feat, gout, idxv, rows, sem):
    wid = _wid()
    base = wid * chunk
    for k in range(nk):
        off = pl.multiple_of(k * npad + base, IB)
        pltpu.sync_copy(idxs.at[pl.ds(off, chunk)], idxv)
        hs = [pltpu.async_copy(feat.at[idxv.at[pl.ds(j * IB, IB)]],
                               rows.at[pl.ds(j * IB, IB), :], sem)
              for j in range(chunk // IB)]
        for h in hs:
            h.wait()
        pltpu.sync_copy(rows, gout.at[pl.ds(base, chunk),
                                      pl.ds(k * c_dim, c_dim)])


def _k4_body(chh, n_real, npad, nk, cap, nbrs, dsts, srcs, cnts,
             nbuf, dbuf, sbuf, cbuf):
    wid = _wid()
    spc = chh // 16
    for v in range(2):
        hw = wid * 2 + v
        hbase = hw * chh

        @pl.loop(0, nk)
        def _(k):
            off = pl.multiple_of(k * npad + hbase, 8)
            koff = pl.multiple_of(k * chh, 8)
            pltpu.sync_copy(nbrs.at[pl.ds(off, chh)],
                            nbuf.at[pl.ds(koff, chh)])

        @pl.loop(0, cap, step=16)
        def _(i):
            dbuf[pl.ds(i, 16)] = jnp.full((16,), chh, jnp.int32)
            sbuf[pl.ds(i, 16)] = jnp.full((16,), n_real, jnp.int32)

        def step(i, cnt):
            vec = nbuf[pl.ds(i * 16, 16)]
            k = i // spc
            slot = (i % spc) * 16
            m = (vec != n_real) & (cnt < cap - 16)
            dv = lax.iota(jnp.int32, 16) + slot
            sv = vec + k * npad
            plsc.store_compressed(dbuf.at[pl.ds(cnt, 16)], dv, mask=m)
            plsc.store_compressed(sbuf.at[pl.ds(cnt, 16)], sv, mask=m)
            return cnt + jnp.sum(m.astype(jnp.int32))

        cnt = lax.fori_loop(0, nk * spc, step, jnp.int32(0))
        cbuf[pl.ds(0, 16)] = jnp.broadcast_to(cnt, (16,))
        pltpu.sync_copy(dbuf, dsts.at[pl.ds(hw * cap, cap)])
        pltpu.sync_copy(sbuf, srcs.at[pl.ds(hw * cap, cap)])
        pltpu.sync_copy(cbuf, cnts.at[pl.ds(hw * 16, 16)])


def _k5_body(chh, c_dim, cap, dsts, srcs, cnts, z, out,
             dbuf, sbuf, cbuf, rows, acc, sem):
    wid = _wid()
    for v in range(2):
        hw = wid * 2 + v
        pltpu.sync_copy(dsts.at[pl.ds(hw * cap, cap)], dbuf.at[pl.ds(0, cap)])
        pltpu.sync_copy(srcs.at[pl.ds(hw * cap, cap)], sbuf)
        pltpu.sync_copy(cnts.at[pl.ds(hw * 16, 16)], cbuf)
        cnt = cbuf[pl.ds(0, 16)][0]

        @pl.loop(0, chh + 8)
        def _(rr):
            @pl.loop(0, c_dim, step=16)
            def _(cc):
                acc[rr, pl.ds(cc, 16)] = jnp.zeros((16,), jnp.float32)

        nb = (cnt + 255) // 256

        def batch(j, _):
            pltpu.async_copy(z.at[sbuf.at[pl.ds(j * 256, 256)]], rows,
                             sem).wait()

            def pstep(p, _):
                d = dbuf[pl.ds(j * 256 + p, 16)][0]
                for cc in range(c_dim // 16):
                    s = pl.ds(cc * 16, 16)
                    acc[d, s] = acc[d, s] + rows[p, s]
                return 0

            lax.fori_loop(0, 256, pstep, 0)
            return 0

        lax.fori_loop(0, nb, batch, 0)
        pltpu.sync_copy(acc.at[pl.ds(0, chh), :],
                        out.at[pl.ds(hw * chh, chh), :])


def _sc_call(body, out_types, scratch, *args, layout_passes=True):
    fn = pl.kernel(body, out_type=out_types, mesh=_mesh,
                   scratch_types=scratch,
                   compiler_params=pltpu.CompilerParams(
                       use_tc_tiling_on_sc=False,
                       needs_layout_passes=layout_passes))
    return fn(*args)


def _bn_kernel(n_real, br, x_ref, s_ref, b_ref, o_ref):
    pid = pl.program_id(0)
    y = jnp.maximum(x_ref[...] * s_ref[...] + b_ref[...], 0.0)
    rows = jax.lax.broadcasted_iota(jnp.int32, y.shape, 0) + pid * br
    o_ref[...] = jnp.where(rows < n_real, y, 0.0)


def _bn_relu_pad(x, g, b, n_real, br=512):
    npad, c = x.shape
    s = (g / np.sqrt(1.0 + EPS)).reshape(1, c)
    b = b.reshape(1, c)
    return pl.pallas_call(
        functools.partial(_bn_kernel, n_real, br),
        grid=(npad // br,),
        in_specs=[pl.BlockSpec((br, c), lambda i: (i, 0)),
                  pl.BlockSpec((1, c), lambda i: (0, 0)),
                  pl.BlockSpec((1, c), lambda i: (0, 0))],
        out_specs=pl.BlockSpec((br, c), lambda i: (i, 0)),
        out_shape=jax.ShapeDtypeStruct((npad, c), jnp.float32),
    )(x, s, b)


def _zmm_kernel(n_real, br, bn, x_ref, w_ref, s_ref, b_ref, o_ref):
    i = pl.program_id(0)
    x = x_ref[...]
    if bn:
        x = jnp.maximum(x * s_ref[...] + b_ref[...], 0.0)
    acc = jnp.dot(x, w_ref[...], preferred_element_type=jnp.float32)
    rows = jax.lax.broadcasted_iota(jnp.int32, acc.shape, 0) + i * br
    o_ref[...] = jnp.where(rows < n_real, acc, 0.0)


def _zmm(xp, wf, nk, n_real, gvec=None, bvec=None, br=512):
    npad, cin = xp.shape
    cout = wf.shape[1]
    nb = npad // br
    bn = gvec is not None
    if gvec is None:
        gvec = jnp.ones((cin,), jnp.float32)
        bvec = jnp.zeros((cin,), jnp.float32)
    s = (gvec / np.sqrt(1.0 + EPS)).reshape(1, cin)
    b = bvec.reshape(1, cin)
    return pl.pallas_call(
        functools.partial(_zmm_kernel, n_real, br, bn),
        grid=(nb, nk),
        in_specs=[pl.BlockSpec((br, cin), lambda i, k: (i, 0)),
                  pl.BlockSpec((cin, cout), lambda i, k: (k, 0)),
                  pl.BlockSpec((1, cin), lambda i, k: (0, 0)),
                  pl.BlockSpec((1, cin), lambda i, k: (0, 0))],
        out_specs=pl.BlockSpec((br, cout), lambda i, k: (k * nb + i, 0)),
        out_shape=jax.ShapeDtypeStruct((nk * npad, cout), jnp.float32),
    )(xp, wf, s, b)


def _up_kernel(n_real, br, u_ref, wu_ref, kx_ref, skip_ref, s_ref, b_ref,
               s4_ref, b4_ref, o_ref):
    pid = pl.program_id(0)
    u = jnp.maximum(u_ref[...] * s4_ref[...] + b4_ref[...], 0.0)
    p = jnp.dot(u, wu_ref[...], preferred_element_type=jnp.float32)
    kx = kx_ref[...]
    up = jnp.zeros((br, 32), jnp.float32)
    for k in range(8):
        sel = (kx == k).astype(jnp.float32)
        up = up + sel * p[:, k * 32:(k + 1) * 32]
    cat = jnp.concatenate([skip_ref[...], up], axis=1)
    y = jnp.maximum(cat * s_ref[...] + b_ref[...], 0.0)
    rows = jax.lax.broadcasted_iota(jnp.int32, y.shape, 0) + pid * br
    o_ref[...] = jnp.where(rows < n_real, y, 0.0)


def _up_concat(u, wuf, kidx, skip, g5, b5, g4, b4, n_real, br=512):
    npad = u.shape[0]
    s = (g5 / np.sqrt(1.0 + EPS)).reshape(1, 64)
    b = b5.reshape(1, 64)
    s4 = (g4 / np.sqrt(1.0 + EPS)).reshape(1, 64)
    b4 = b4.reshape(1, 64)
    return pl.pallas_call(
        functools.partial(_up_kernel, n_real, br),
        grid=(npad // br,),
        in_specs=[pl.BlockSpec((br, 64), lambda i: (i, 0)),
                  pl.BlockSpec((64, 256), lambda i: (0, 0)),
                  pl.BlockSpec((br, 1), lambda i: (i, 0)),
                  pl.BlockSpec((br, 32), lambda i: (i, 0)),
                  pl.BlockSpec((1, 64), lambda i: (0, 0)),
                  pl.BlockSpec((1, 64), lambda i: (0, 0)),
                  pl.BlockSpec((1, 64), lambda i: (0, 0)),
                  pl.BlockSpec((1, 64), lambda i: (0, 0))],
        out_specs=pl.BlockSpec((br, 64), lambda i: (i, 0)),
        out_shape=jax.ShapeDtypeStruct((npad, 64), jnp.float32),
    )(u, wuf, kidx, skip, s, b, s4, b4)


def kernel(feat, pos, W1, Wd, W2, Wu, W3,
           g1, b1, g2, b2, g3, b3, g4, b4, g5, b5):
    n = feat.shape[0]
    chunk = ((n + 1 + NW - 1) // NW + IB - 1) // IB * IB
    npad = NW * chunk
    chh = chunk // 2
    nw2 = NW * 2

    i32 = jnp.int32
    posp = jnp.concatenate(
        [pos.astype(i32), jnp.full((npad - n, 3), -2, i32)], 0)
    posx = posp[:, 0] * 1
    posy = posp[:, 1] * 1
    posz = posp[:, 2] * 1
    featp = jnp.concatenate(
        [feat, jnp.zeros((npad - n, 32), feat.dtype)], 0)

    sd = jax.ShapeDtypeStruct
    vm = pltpu.VMEM

    t0, t1, poskey, k1key = _sc_call(
        functools.partial(_k1_body, chunk),
        (sd((T0SIZE,), i32), sd((T1SIZE,), i32),
         sd((npad,), i32), sd((npad,), i32)),
        [vm((chunk,), i32)] * 6 + [pltpu.SemaphoreType.DMA],
        posx, posy, posz)

    nbr0, dg, nbr1, kidx = _sc_call(
        functools.partial(_k2_body, chunk, n, npad),
        (sd((27 * npad,), i32), sd((8 * npad,), i32),
         sd((27 * npad,), i32), sd((npad,), i32)),
        [vm((chunk,), i32)] * 11 + [pltpu.SemaphoreType.DMA],
        posx, posy, posz, t0, t1, poskey, k1key)

    def compact(nbrs, nk, cap):
        return _sc_call(
            functools.partial(_k4_body, chh, n, npad, nk, cap),
            (sd((nw2 * cap,), i32), sd((nw2 * cap,), i32),
             sd((nw2 * 16,), i32)),
            [vm((nk * chh,), i32), vm((cap,), i32), vm((cap,), i32),
             vm((16,), i32)],
            nbrs, layout_passes=False), cap

    def sconv(pairs, z, c):
        (dsts, srcs, cnts), cap = pairs
        return _sc_call(
            functools.partial(_k5_body, chh, c, cap),
            sd((npad, c), jnp.float32),
            [vm((cap + 16,), i32), vm((cap,), i32), vm((16,), i32),
             vm((256, c), jnp.float32), vm((chh + 8, c), jnp.float32),
             pltpu.SemaphoreType.DMA],
            dsts, srcs, cnts, z)

    def gather(idxs, f, nk, c):
        return _sc_call(
            functools.partial(_k3_body, chunk, npad, nk, c),
            sd((npad, nk * c), jnp.float32),
            [vm((chunk,), i32), vm((chunk, c), jnp.float32),
             pltpu.SemaphoreType.DMA],
            idxs, f)

    w1f = W1.reshape(27 * 32, 32)
    wdf = Wd.reshape(8 * 32, 64)
    w2f = W2.reshape(27 * 64, 64)
    wuf = Wu.transpose(1, 0, 2).reshape(64, 8 * 32)
    w3f = W3.reshape(27 * 64, 32)

    pairs0 = compact(nbr0, 27, 4096)
    pairsd = compact(dg, 8, 2048)
    pairs1 = compact(nbr1, 27, 8192)

    x_raw = sconv(pairs0, _zmm(featp, w1f, 27, n, g1, b1), 32)
    xd = sconv(pairsd, _zmm(x_raw, wdf, 8, n, g2, b2), 64)
    y_raw = sconv(pairs1, _zmm(xd, w2f, 27, n, g3, b3), 64)
    um = gather(nbr1[13 * npad:14 * npad], y_raw, 1, 64)
    x3p = _up_concat(um, wuf, kidx.reshape(npad, 1), x_raw,
                     g5, b5, g4, b4, n)
    out = sconv(pairs0, _zmm(x3p, w3f, 27, n), 32)
    return out[:n]


# speedup vs baseline: 29.0243x; 1.2519x over previous
import functools
import numpy as np
import jax
import jax.numpy as jnp
from jax import lax
from jax.experimental import pallas as pl
from jax.experimental.pallas import tpu as pltpu
from jax.experimental.pallas import tpu_sc as plsc

EPS = 1e-4
B0 = 136
B1 = 68
T0SIZE = 2424832
T1SIZE = 309760

NC, NS = 2, 16
NW = NC * NS
IB = 128

_mesh = plsc.VectorSubcoreMesh(core_axis_name="c", subcore_axis_name="s")


def _wid():
    return lax.axis_index("s") * NC + lax.axis_index("c")


def _k1_body(chunk, posx, posy, posz, t0, t1, poskey, k1key,
             px, py, pz, k0v, k1v, valv, sem):
    wid = _wid()
    base = wid * chunk
    pltpu.sync_copy(posx.at[pl.ds(base, chunk)], px)
    pltpu.sync_copy(posy.at[pl.ds(base, chunk)], py)
    pltpu.sync_copy(posz.at[pl.ds(base, chunk)], pz)

    @pl.loop(0, chunk, step=16)
    def _(i):
        s = pl.ds(i, 16)
        x, y, z = px[s], py[s], pz[s]
        k0v[s] = ((x + 2) * B0 + (y + 2)) * B0 + (z + 2)
        x1, y1, z1 = x >> 1, y >> 1, z >> 1
        k1v[s] = ((x1 + 2) * B1 + (y1 + 2)) * B1 + (z1 + 2)
        valv[s] = lax.iota(jnp.int32, 16) + (base + i)

    pltpu.sync_copy(k0v, poskey.at[pl.ds(base, chunk)])
    pltpu.sync_copy(k1v, k1key.at[pl.ds(base, chunk)])
    hs = []
    for j in range(chunk // IB):
        s = pl.ds(j * IB, IB)
        hs.append(pltpu.async_copy(valv.at[s], t0.at[k0v.at[s]], sem))
        hs.append(pltpu.async_copy(valv.at[s], t1.at[k1v.at[s]], sem))
    for h in hs:
        h.wait()


def _k2_body(chunk, n_real, npad, posx, posy, posz, t0, t1, poskey, k1key,
             nbr0, dg, nbr1, kidx,
             px, py, pz, k0v, kev, k1v, qkv, cv, nbv, vsl, vcc, vqk, vkb,
             sem):
    wid = _wid()
    base = wid * chunk
    pltpu.sync_copy(posx.at[pl.ds(base, chunk)], px)
    pltpu.sync_copy(posy.at[pl.ds(base, chunk)], py)
    pltpu.sync_copy(posz.at[pl.ds(base, chunk)], pz)

    @pl.loop(0, chunk + 16, step=16)
    def _(i):
        vcc[pl.ds(i, 16)] = jnp.zeros((16,), jnp.int32)

    @pl.loop(0, chunk, step=16)
    def _(i):
        s = pl.ds(i, 16)
        x, y, z = px[s], py[s], pz[s]
        k0v[s] = ((x + 2) * B0 + (y + 2)) * B0 + (z + 2)
        ex, ey, ez = x & -2, y & -2, z & -2
        kev[s] = ((ex + 2) * B0 + (ey + 2)) * B0 + (ez + 2)
        x1, y1, z1 = x >> 1, y >> 1, z >> 1
        k1v[s] = ((x1 + 2) * B1 + (y1 + 2)) * B1 + (z1 + 2)
        nbv[s] = (x & 1) * 4 + (y & 1) * 2 + (z & 1)

    pltpu.sync_copy(nbv, kidx.at[pl.ds(base, chunk)])

    def lookup(keyv, dk, tab, karr, out_ref, k, tsize):
        @pl.loop(0, chunk, step=16)
        def _(i):
            s = pl.ds(i, 16)
            q = keyv[s] + dk
            qkv[s] = jnp.minimum(jnp.maximum(q, 0), tsize - 1)

        hs = [pltpu.async_copy(tab.at[qkv.at[pl.ds(j * IB, IB)]],
                               cv.at[pl.ds(j * IB, IB)], sem)
              for j in range(chunk // IB)]
        for h in hs:
            h.wait()

        def c_step(i, nv):
            s = pl.ds(i * 16, 16)
            c = cv[s]
            m = (c >= 0) & (c < n_real)
            plsc.store_compressed(vsl.at[pl.ds(nv, 16)],
                                  lax.iota(jnp.int32, 16) + i * 16, mask=m)
            plsc.store_compressed(vcc.at[pl.ds(nv, 16)], c, mask=m)
            plsc.store_compressed(vqk.at[pl.ds(nv, 16)], qkv[s], mask=m)
            return nv + jnp.sum(m.astype(jnp.int32))

        nv = lax.fori_loop(0, chunk // 16, c_step, jnp.int32(0))

        def g_step(j, _):
            pltpu.async_copy(karr.at[vcc.at[pl.ds(j * IB, IB)]],
                             vkb.at[pl.ds(j * IB, IB)], sem).wait()
            return 0

        lax.fori_loop(0, (nv + IB - 1) // IB, g_step, 0)

        @pl.loop(0, chunk, step=16)
        def _(i):
            nbv[pl.ds(i, 16)] = jnp.full((16,), n_real, jnp.int32)

        def s_step(i, _):
            s = pl.ds(i * 16, 16)
            lane = lax.iota(jnp.int32, 16) + i * 16
            ok = (vkb[s] == vqk[s]) & (lane < nv)
            plsc.store_scatter(nbv, [vsl[s]], vcc[s], mask=ok)
            return 0

        lax.fori_loop(0, (nv + 15) // 16, s_step, 0)

        off = pl.multiple_of(k * npad + base, IB)
        pltpu.sync_copy(nbv, out_ref.at[pl.ds(off, chunk)])

    @pl.loop(0, 27)
    def _(k):
        di, dj, dl = k // 9 - 1, (k // 3) % 3 - 1, k % 3 - 1
        lookup(k0v, (di * B0 + dj) * B0 + dl, t0, poskey, nbr0, k, T0SIZE)

    @pl.loop(0, 8)
    def _(k):
        di, dj, dl = k // 4, (k // 2) % 2, k % 2
        lookup(kev, (di * B0 + dj) * B0 + dl, t0, poskey, dg, k, T0SIZE)

    @pl.loop(0, 27)
    def _(k):
        di, dj, dl = k // 9 - 1, (k // 3) % 3 - 1, k % 3 - 1
        lookup(k1v, (di * B1 + dj) * B1 + dl, t1, k1key, nbr1, k, T1SIZE)


def _k3_body(chunk, npad, nk, c_dim, idxs, feat, gout, idxv, rows, sem):
    wid = _wid()
    base = wid * chunk
    for k in range(nk):
        off = pl.multiple_of(k * npad + base, IB)
        pltpu.sync_copy(idxs.at[pl.ds(off, chunk)], idxv)
        hs = [pltpu.async_copy(feat.at[idxv.at[pl.ds(j * IB, IB)]],
                               rows.at[pl.ds(j * IB, IB), :], sem)
              for j in range(chunk // IB)]
        for h in hs:
            h.wait()
        pltpu.sync_copy(rows, gout.at[pl.ds(base, chunk),
                                      pl.ds(k * c_dim, c_dim)])


def _k4_body(chh, n_real, npad, nk, cap, nbrs, dsts, srcs, cnts,
             nbuf, dbuf, sbuf, cbuf):
    wid = _wid()
    spc = chh // 16
    for v in range(2):
        hw = wid * 2 + v
        hbase = hw * chh

        @pl.loop(0, nk)
        def _(k):
            off = pl.multiple_of(k * npad + hbase, 8)
            koff = pl.multiple_of(k * chh, 8)
            pltpu.sync_copy(nbrs.at[pl.ds(off, chh)],
                            nbuf.at[pl.ds(koff, chh)])

        @pl.loop(0, cap, step=16)
        def _(i):
            dbuf[pl.ds(i, 16)] = jnp.full((16,), chh, jnp.int32)
            sbuf[pl.ds(i, 16)] = jnp.full((16,), n_real, jnp.int32)

        def step(i, cnt):
            vec = nbuf[pl.ds(i * 16, 16)]
            k = i // spc
            slot = (i % spc) * 16
            m = (vec != n_real) & (cnt < cap - 16)
            dv = lax.iota(jnp.int32, 16) + slot
            sv = vec + k * npad
            plsc.store_compressed(dbuf.at[pl.ds(cnt, 16)], dv, mask=m)
            plsc.store_compressed(sbuf.at[pl.ds(cnt, 16)], sv, mask=m)
            return cnt + jnp.sum(m.astype(jnp.int32))

        cnt = lax.fori_loop(0, nk * spc, step, jnp.int32(0))
        cbuf[pl.ds(0, 16)] = jnp.broadcast_to(cnt, (16,))
        pltpu.sync_copy(dbuf, dsts.at[pl.ds(hw * cap, cap)])
        pltpu.sync_copy(sbuf, srcs.at[pl.ds(hw * cap, cap)])
        pltpu.sync_copy(cbuf, cnts.at[pl.ds(hw * 16, 16)])


def _k5_body(chh, c_dim, cap, dsts, srcs, cnts, z, out,
             dbuf, sbuf, cbuf, rows, acc, sem):
    wid = _wid()
    for v in range(2):
        hw = wid * 2 + v
        pltpu.sync_copy(dsts.at[pl.ds(hw * cap, cap)], dbuf.at[pl.ds(0, cap)])
        pltpu.sync_copy(srcs.at[pl.ds(hw * cap, cap)], sbuf)
        pltpu.sync_copy(cnts.at[pl.ds(hw * 16, 16)], cbuf)
        cnt = cbuf[pl.ds(0, 16)][0]

        @pl.loop(0, chh + 8)
        def _(rr):
            @pl.loop(0, c_dim, step=16)
            def _(cc):
                acc[rr, pl.ds(cc, 16)] = jnp.zeros((16,), jnp.float32)

        nb = (cnt + 255) // 256

        def batch(j, _):
            pltpu.async_copy(z.at[sbuf.at[pl.ds(j * 256, 256)]], rows,
                             sem).wait()

            def pstep(p, _):
                d = dbuf[pl.ds(j * 256 + p, 16)][0]
                for cc in range(c_dim // 16):
                    s = pl.ds(cc * 16, 16)
                    acc[d, s] = acc[d, s] + rows[p, s]
                return 0

            lax.fori_loop(0, 256, pstep, 0)
            return 0

        lax.fori_loop(0, nb, batch, 0)
        pltpu.sync_copy(acc.at[pl.ds(0, chh), :],
                        out.at[pl.ds(hw * chh, chh), :])


def _sc_call(body, out_types, scratch, *args, layout_passes=True):
    fn = pl.kernel(body, out_type=out_types, mesh=_mesh,
                   scratch_types=scratch,
                   compiler_params=pltpu.CompilerParams(
                       use_tc_tiling_on_sc=False,
                       needs_layout_passes=layout_passes))
    return fn(*args)


def _bn_kernel(n_real, br, x_ref, s_ref, b_ref, o_ref):
    pid = pl.program_id(0)
    y = jnp.maximum(x_ref[...] * s_ref[...] + b_ref[...], 0.0)
    rows = jax.lax.broadcasted_iota(jnp.int32, y.shape, 0) + pid * br
    o_ref[...] = jnp.where(rows < n_real, y, 0.0)


def _bn_relu_pad(x, g, b, n_real, br=512):
    npad, c = x.shape
    s = (g / np.sqrt(1.0 + EPS)).reshape(1, c)
    b = b.reshape(1, c)
    return pl.pallas_call(
        functools.partial(_bn_kernel, n_real, br),
        grid=(npad // br,),
        in_specs=[pl.BlockSpec((br, c), lambda i: (i, 0)),
                  pl.BlockSpec((1, c), lambda i: (0, 0)),
                  pl.BlockSpec((1, c), lambda i: (0, 0))],
        out_specs=pl.BlockSpec((br, c), lambda i: (i, 0)),
        out_shape=jax.ShapeDtypeStruct((npad, c), jnp.float32),
    )(x, s, b)


def _zmm_kernel(n_real, br, bn, x_ref, w_ref, s_ref, b_ref, o_ref):
    i = pl.program_id(0)
    x = x_ref[...]
    if bn:
        x = jnp.maximum(x * s_ref[...] + b_ref[...], 0.0)
    acc = jnp.dot(x, w_ref[...], preferred_element_type=jnp.float32)
    rows = jax.lax.broadcasted_iota(jnp.int32, acc.shape, 0) + i * br
    o_ref[...] = jnp.where(rows < n_real, acc, 0.0)


def _zmm(xp, wf, nk, n_real, gvec=None, bvec=None, br=512):
    npad, cin = xp.shape
    cout = wf.shape[1]
    nb = npad // br
    bn = gvec is not None
    if gvec is None:
        gvec = jnp.ones((cin,), jnp.float32)
        bvec = jnp.zeros((cin,), jnp.float32)
    s = (gvec / np.sqrt(1.0 + EPS)).reshape(1, cin)
    b = bvec.reshape(1, cin)
    return pl.pallas_call(
        functools.partial(_zmm_kernel, n_real, br, bn),
        grid=(nb, nk),
        in_specs=[pl.BlockSpec((br, cin), lambda i, k: (i, 0)),
                  pl.BlockSpec((cin, cout), lambda i, k: (k, 0)),
                  pl.BlockSpec((1, cin), lambda i, k: (0, 0)),
                  pl.BlockSpec((1, cin), lambda i, k: (0, 0))],
        out_specs=pl.BlockSpec((br, cout), lambda i, k: (k * nb + i, 0)),
        out_shape=jax.ShapeDtypeStruct((nk * npad, cout), jnp.float32),
    )(xp, wf, s, b)


def _up_kernel(n_real, br, u_ref, wu_ref, kx_ref, skip_ref, s_ref, b_ref,
               s4_ref, b4_ref, o_ref):
    pid = pl.program_id(0)
    u = jnp.maximum(u_ref[...] * s4_ref[...] + b4_ref[...], 0.0)
    p = jnp.dot(u, wu_ref[...], preferred_element_type=jnp.float32)
    kx = kx_ref[...]
    up = jnp.zeros((br, 32), jnp.float32)
    for k in range(8):
        sel = (kx == k).astype(jnp.float32)
        up = up + sel * p[:, k * 32:(k + 1) * 32]
    cat = jnp.concatenate([skip_ref[...], up], axis=1)
    y = jnp.maximum(cat * s_ref[...] + b_ref[...], 0.0)
    rows = jax.lax.broadcasted_iota(jnp.int32, y.shape, 0) + pid * br
    o_ref[...] = jnp.where(rows < n_real, y, 0.0)


def _up_concat(u, wuf, kidx, skip, g5, b5, g4, b4, n_real, br=512):
    npad = u.shape[0]
    s = (g5 / np.sqrt(1.0 + EPS)).reshape(1, 64)
    b = b5.reshape(1, 64)
    s4 = (g4 / np.sqrt(1.0 + EPS)).reshape(1, 64)
    b4 = b4.reshape(1, 64)
    return pl.pallas_call(
        functools.partial(_up_kernel, n_real, br),
        grid=(npad // br,),
        in_specs=[pl.BlockSpec((br, 64), lambda i: (i, 0)),
                  pl.BlockSpec((64, 256), lambda i: (0, 0)),
                  pl.BlockSpec((br, 1), lambda i: (i, 0)),
                  pl.BlockSpec((br, 32), lambda i: (i, 0)),
                  pl.BlockSpec((1, 64), lambda i: (0, 0)),
                  pl.BlockSpec((1, 64), lambda i: (0, 0)),
                  pl.BlockSpec((1, 64), lambda i: (0, 0)),
                  pl.BlockSpec((1, 64), lambda i: (0, 0))],
        out_specs=pl.BlockSpec((br, 64), lambda i: (i, 0)),
        out_shape=jax.ShapeDtypeStruct((npad, 64), jnp.float32),
    )(u, wuf, kidx, skip, s, b, s4, b4)


def kernel(feat, pos, W1, Wd, W2, Wu, W3,
           g1, b1, g2, b2, g3, b3, g4, b4, g5, b5):
    n = feat.shape[0]
    chunk = ((n + 1 + NW - 1) // NW + IB - 1) // IB * IB
    npad = NW * chunk
    chh = chunk // 2
    nw2 = NW * 2

    i32 = jnp.int32
    posp = jnp.concatenate(
        [pos.astype(i32), jnp.full((npad - n, 3), -2, i32)], 0)
    posx = posp[:, 0] * 1
    posy = posp[:, 1] * 1
    posz = posp[:, 2] * 1
    featp = jnp.concatenate(
        [feat, jnp.zeros((npad - n, 32), feat.dtype)], 0)

    sd = jax.ShapeDtypeStruct
    vm = pltpu.VMEM

    t0, t1, poskey, k1key = _sc_call(
        functools.partial(_k1_body, chunk),
        (sd((T0SIZE,), i32), sd((T1SIZE,), i32),
         sd((npad,), i32), sd((npad,), i32)),
        [vm((chunk,), i32)] * 6 + [pltpu.SemaphoreType.DMA],
        posx, posy, posz)

    nbr0, dg, nbr1, kidx = _sc_call(
        functools.partial(_k2_body, chunk, n, npad),
        (sd((27 * npad,), i32), sd((8 * npad,), i32),
         sd((27 * npad,), i32), sd((npad,), i32)),
        [vm((chunk,), i32)] * 9 + [vm((chunk + 16,), i32)] * 4
        + [pltpu.SemaphoreType.DMA],
        posx, posy, posz, t0, t1, poskey, k1key, layout_passes=False)

    def compact(nbrs, nk, cap):
        return _sc_call(
            functools.partial(_k4_body, chh, n, npad, nk, cap),
            (sd((nw2 * cap,), i32), sd((nw2 * cap,), i32),
             sd((nw2 * 16,), i32)),
            [vm((nk * chh,), i32), vm((cap,), i32), vm((cap,), i32),
             vm((16,), i32)],
            nbrs, layout_passes=False), cap

    def sconv(pairs, z, c):
        (dsts, srcs, cnts), cap = pairs
        return _sc_call(
            functools.partial(_k5_body, chh, c, cap),
            sd((npad, c), jnp.float32),
            [vm((cap + 16,), i32), vm((cap,), i32), vm((16,), i32),
             vm((256, c), jnp.float32), vm((chh + 8, c), jnp.float32),
             pltpu.SemaphoreType.DMA],
            dsts, srcs, cnts, z)

    def gather(idxs, f, nk, c):
        return _sc_call(
            functools.partial(_k3_body, chunk, npad, nk, c),
            sd((npad, nk * c), jnp.float32),
            [vm((chunk,), i32), vm((chunk, c), jnp.float32),
             pltpu.SemaphoreType.DMA],
            idxs, f)

    w1f = W1.reshape(27 * 32, 32)
    wdf = Wd.reshape(8 * 32, 64)
    w2f = W2.reshape(27 * 64, 64)
    wuf = Wu.transpose(1, 0, 2).reshape(64, 8 * 32)
    w3f = W3.reshape(27 * 64, 32)

    pairs0 = compact(nbr0, 27, 4096)
    pairsd = compact(dg, 8, 2048)
    pairs1 = compact(nbr1, 27, 8192)

    x_raw = sconv(pairs0, _zmm(featp, w1f, 27, n, g1, b1), 32)
    xd = sconv(pairsd, _zmm(x_raw, wdf, 8, n, g2, b2), 64)
    y_raw = sconv(pairs1, _zmm(xd, w2f, 27, n, g3, b3), 64)
    um = gather(nbr1[13 * npad:14 * npad], y_raw, 1, 64)
    x3p = _up_concat(um, wuf, kidx.reshape(npad, 1), x_raw,
                     g5, b5, g4, b4, n)
    out = sconv(pairs0, _zmm(x3p, w3f, 27, n), 32)
    return out[:n]


# speedup vs baseline: 38.1548x; 1.3146x over previous
import functools
import numpy as np
import jax
import jax.numpy as jnp
from jax import lax
from jax.experimental import pallas as pl
from jax.experimental.pallas import tpu as pltpu
from jax.experimental.pallas import tpu_sc as plsc

EPS = 1e-4
B0 = 136
B1 = 68
T0SIZE = 2424832
T1SIZE = 309760

NC, NS = 2, 16
NW = NC * NS
IB = 128

_mesh = plsc.VectorSubcoreMesh(core_axis_name="c", subcore_axis_name="s")


def _wid():
    return lax.axis_index("s") * NC + lax.axis_index("c")


def _k1_body(chunk, posx, posy, posz, t0, t1, poskey, k1key,
             px, py, pz, k0v, k1v, valv, sem):
    wid = _wid()
    base = wid * chunk
    pltpu.sync_copy(posx.at[pl.ds(base, chunk)], px)
    pltpu.sync_copy(posy.at[pl.ds(base, chunk)], py)
    pltpu.sync_copy(posz.at[pl.ds(base, chunk)], pz)

    @pl.loop(0, chunk, step=16)
    def _(i):
        s = pl.ds(i, 16)
        x, y, z = px[s], py[s], pz[s]
        k0v[s] = ((x + 2) * B0 + (y + 2)) * B0 + (z + 2)
        x1, y1, z1 = x >> 1, y >> 1, z >> 1
        k1v[s] = ((x1 + 2) * B1 + (y1 + 2)) * B1 + (z1 + 2)
        valv[s] = lax.iota(jnp.int32, 16) + (base + i)

    pltpu.sync_copy(k0v, poskey.at[pl.ds(base, chunk)])
    pltpu.sync_copy(k1v, k1key.at[pl.ds(base, chunk)])
    hs = []
    for j in range(chunk // IB):
        s = pl.ds(j * IB, IB)
        hs.append(pltpu.async_copy(valv.at[s], t0.at[k0v.at[s]], sem))
        hs.append(pltpu.async_copy(valv.at[s], t1.at[k1v.at[s]], sem))
    for h in hs:
        h.wait()


def _k2_body(chunk, n_real, npad, posx, posy, posz, t0, t1, poskey, k1key,
             nbr0, dg, nbr1, kidx,
             px, py, pz, k0v, kev, k1v, qkv, cv, nbv, vsl, vcc, vqk, vkb,
             sem):
    wid = _wid()
    base = wid * chunk
    pltpu.sync_copy(posx.at[pl.ds(base, chunk)], px)
    pltpu.sync_copy(posy.at[pl.ds(base, chunk)], py)
    pltpu.sync_copy(posz.at[pl.ds(base, chunk)], pz)

    @pl.loop(0, chunk + 16, step=16)
    def _(i):
        vcc[pl.ds(i, 16)] = jnp.zeros((16,), jnp.int32)

    @pl.loop(0, chunk, step=16)
    def _(i):
        s = pl.ds(i, 16)
        x, y, z = px[s], py[s], pz[s]
        k0v[s] = ((x + 2) * B0 + (y + 2)) * B0 + (z + 2)
        ex, ey, ez = x & -2, y & -2, z & -2
        kev[s] = ((ex + 2) * B0 + (ey + 2)) * B0 + (ez + 2)
        x1, y1, z1 = x >> 1, y >> 1, z >> 1
        k1v[s] = ((x1 + 2) * B1 + (y1 + 2)) * B1 + (z1 + 2)
        nbv[s] = (x & 1) * 4 + (y & 1) * 2 + (z & 1)

    pltpu.sync_copy(nbv, kidx.at[pl.ds(base, chunk)])

    def lookup(keyv, dk, tab, karr, out_ref, k, tsize):
        @pl.loop(0, chunk, step=16)
        def _(i):
            s = pl.ds(i, 16)
            q = keyv[s] + dk
            qkv[s] = jnp.minimum(jnp.maximum(q, 0), tsize - 1)

        hs = [pltpu.async_copy(tab.at[qkv.at[pl.ds(j * IB, IB)]],
                               cv.at[pl.ds(j * IB, IB)], sem)
              for j in range(chunk // IB)]
        for h in hs:
            h.wait()

        def c_step(i, nv):
            s = pl.ds(i * 16, 16)
            c = cv[s]
            m = (c >= 0) & (c < n_real)
            plsc.store_compressed(vsl.at[pl.ds(nv, 16)],
                                  lax.iota(jnp.int32, 16) + i * 16, mask=m)
            plsc.store_compressed(vcc.at[pl.ds(nv, 16)], c, mask=m)
            plsc.store_compressed(vqk.at[pl.ds(nv, 16)], qkv[s], mask=m)
            return nv + jnp.sum(m.astype(jnp.int32))

        nv = lax.fori_loop(0, chunk // 16, c_step, jnp.int32(0))

        def g_step(j, _):
            pltpu.async_copy(karr.at[vcc.at[pl.ds(j * IB, IB)]],
                             vkb.at[pl.ds(j * IB, IB)], sem).wait()
            return 0

        lax.fori_loop(0, (nv + IB - 1) // IB, g_step, 0)

        @pl.loop(0, chunk, step=16)
        def _(i):
            nbv[pl.ds(i, 16)] = jnp.full((16,), n_real, jnp.int32)

        def s_step(i, _):
            s = pl.ds(i * 16, 16)
            lane = lax.iota(jnp.int32, 16) + i * 16
            ok = (vkb[s] == vqk[s]) & (lane < nv)
            plsc.store_scatter(nbv, [vsl[s]], vcc[s], mask=ok)
            return 0

        lax.fori_loop(0, (nv + 15) // 16, s_step, 0)

        off = pl.multiple_of(k * npad + base, IB)
        pltpu.sync_copy(nbv, out_ref.at[pl.ds(off, chunk)])

    @pl.loop(0, 27)
    def _(k):
        di, dj, dl = k // 9 - 1, (k // 3) % 3 - 1, k % 3 - 1
        lookup(k0v, (di * B0 + dj) * B0 + dl, t0, poskey, nbr0, k, T0SIZE)

    @pl.loop(0, 8)
    def _(k):
        di, dj, dl = k // 4, (k // 2) % 2, k % 2
        lookup(kev, (di * B0 + dj) * B0 + dl, t0, poskey, dg, k, T0SIZE)

    @pl.loop(0, 27)
    def _(k):
        di, dj, dl = k // 9 - 1, (k // 3) % 3 - 1, k % 3 - 1
        lookup(k1v, (di * B1 + dj) * B1 + dl, t1, k1key, nbr1, k, T1SIZE)


def _k3_body(chunk, npad, nk, c_dim, idxs, feat, gout, idxv, rows, sem):
    wid = _wid()
    base = wid * chunk
    for k in range(nk):
        off = pl.multiple_of(k * npad + base, IB)
        pltpu.sync_copy(idxs.at[pl.ds(off, chunk)], idxv)
        hs = [pltpu.async_copy(feat.at[idxv.at[pl.ds(j * IB, IB)]],
                               rows.at[pl.ds(j * IB, IB), :], sem)
              for j in range(chunk // IB)]
        for h in hs:
            h.wait()
        pltpu.sync_copy(rows, gout.at[pl.ds(base, chunk),
                                      pl.ds(k * c_dim, c_dim)])


def _k4_one(chh, n_real, npad, nk, cap, nbrs, dsts, srcs, cnts,
            nbuf, dbuf, sbuf, cbuf):
    wid = _wid()
    spc = chh // 16
    for v in range(2):
        hw = wid * 2 + v
        hbase = hw * chh

        @pl.loop(0, nk)
        def _(k):
            off = pl.multiple_of(k * npad + hbase, 8)
            koff = pl.multiple_of(k * chh, 8)
            pltpu.sync_copy(nbrs.at[pl.ds(off, chh)],
                            nbuf.at[pl.ds(koff, chh)])

        @pl.loop(0, cap, step=16)
        def _(i):
            dbuf[pl.ds(i, 16)] = jnp.full((16,), chh, jnp.int32)
            sbuf[pl.ds(i, 16)] = jnp.full((16,), n_real, jnp.int32)

        def step(i, cnt):
            vec = nbuf[pl.ds(i * 16, 16)]
            k = i // spc
            slot = (i % spc) * 16
            m = (vec != n_real) & (cnt < cap - 16)
            dv = lax.iota(jnp.int32, 16) + slot
            sv = vec + k * npad
            plsc.store_compressed(dbuf.at[pl.ds(cnt, 16)], dv, mask=m)
            plsc.store_compressed(sbuf.at[pl.ds(cnt, 16)], sv, mask=m)
            return cnt + jnp.sum(m.astype(jnp.int32))

        cnt = lax.fori_loop(0, nk * spc, step, jnp.int32(0))
        cbuf[pl.ds(0, 16)] = jnp.broadcast_to(cnt, (16,))
        pltpu.sync_copy(dbuf.at[pl.ds(0, cap)], dsts.at[pl.ds(hw * cap, cap)])
        pltpu.sync_copy(sbuf.at[pl.ds(0, cap)], srcs.at[pl.ds(hw * cap, cap)])
        pltpu.sync_copy(cbuf, cnts.at[pl.ds(hw * 16, 16)])


def _k4_body(chh, n_real, npad, caps, nbr0, dg, nbr1,
             d0, s0, c0, dd, sd_, cd, d1, s1, c1,
             nbuf, dbuf, sbuf, cbuf):
    cap0, capd, cap1 = caps
    _k4_one(chh, n_real, npad, 27, cap0, nbr0, d0, s0, c0,
            nbuf, dbuf, sbuf, cbuf)
    _k4_one(chh, n_real, npad, 8, capd, dg, dd, sd_, cd,
            nbuf, dbuf, sbuf, cbuf)
    _k4_one(chh, n_real, npad, 27, cap1, nbr1, d1, s1, c1,
            nbuf, dbuf, sbuf, cbuf)


def _k5_body(chh, c_dim, cap, dsts, srcs, cnts, z, out,
             dbuf, sbuf, cbuf, rows, acc, sem):
    wid = _wid()
    for v in range(2):
        hw = wid * 2 + v
        pltpu.sync_copy(dsts.at[pl.ds(hw * cap, cap)], dbuf.at[pl.ds(0, cap)])
        pltpu.sync_copy(srcs.at[pl.ds(hw * cap, cap)], sbuf)
        pltpu.sync_copy(cnts.at[pl.ds(hw * 16, 16)], cbuf)
        cnt = cbuf[pl.ds(0, 16)][0]

        @pl.loop(0, chh + 8)
        def _(rr):
            @pl.loop(0, c_dim, step=16)
            def _(cc):
                acc[rr, pl.ds(cc, 16)] = jnp.zeros((16,), jnp.float32)

        nb = (cnt + 255) // 256

        def batch(j, _):
            pltpu.async_copy(z.at[sbuf.at[pl.ds(j * 256, 256)]], rows,
                             sem).wait()

            def pstep(p, _):
                d = dbuf[pl.ds(j * 256 + p, 16)][0]
                for cc in range(c_dim // 16):
                    s = pl.ds(cc * 16, 16)
                    acc[d, s] = acc[d, s] + rows[p, s]
                return 0

            lax.fori_loop(0, 256, pstep, 0)
            return 0

        lax.fori_loop(0, nb, batch, 0)
        pltpu.sync_copy(acc.at[pl.ds(0, chh), :],
                        out.at[pl.ds(hw * chh, chh), :])


def _sc_call(body, out_types, scratch, *args, layout_passes=True):
    fn = pl.kernel(body, out_type=out_types, mesh=_mesh,
                   scratch_types=scratch,
                   compiler_params=pltpu.CompilerParams(
                       use_tc_tiling_on_sc=False,
                       needs_layout_passes=layout_passes))
    return fn(*args)


def _bn_kernel(n_real, br, x_ref, s_ref, b_ref, o_ref):
    pid = pl.program_id(0)
    y = jnp.maximum(x_ref[...] * s_ref[...] + b_ref[...], 0.0)
    rows = jax.lax.broadcasted_iota(jnp.int32, y.shape, 0) + pid * br
    o_ref[...] = jnp.where(rows < n_real, y, 0.0)


def _bn_relu_pad(x, g, b, n_real, br=512):
    npad, c = x.shape
    s = (g / np.sqrt(1.0 + EPS)).reshape(1, c)
    b = b.reshape(1, c)
    return pl.pallas_call(
        functools.partial(_bn_kernel, n_real, br),
        grid=(npad // br,),
        in_specs=[pl.BlockSpec((br, c), lambda i: (i, 0)),
                  pl.BlockSpec((1, c), lambda i: (0, 0)),
                  pl.BlockSpec((1, c), lambda i: (0, 0))],
        out_specs=pl.BlockSpec((br, c), lambda i: (i, 0)),
        out_shape=jax.ShapeDtypeStruct((npad, c), jnp.float32),
    )(x, s, b)


def _zmm_kernel(n_real, br, bn, x_ref, w_ref, s_ref, b_ref, o_ref):
    i = pl.program_id(0)
    x = x_ref[...]
    if bn:
        x = jnp.maximum(x * s_ref[...] + b_ref[...], 0.0)
    acc = jnp.dot(x, w_ref[...], preferred_element_type=jnp.float32)
    rows = jax.lax.broadcasted_iota(jnp.int32, acc.shape, 0) + i * br
    o_ref[...] = jnp.where(rows < n_real, acc, 0.0)


def _zmm(xp, wf, nk, n_real, gvec=None, bvec=None, br=512):
    npad, cin = xp.shape
    cout = wf.shape[1]
    nb = npad // br
    bn = gvec is not None
    if gvec is None:
        gvec = jnp.ones((cin,), jnp.float32)
        bvec = jnp.zeros((cin,), jnp.float32)
    s = (gvec / np.sqrt(1.0 + EPS)).reshape(1, cin)
    b = bvec.reshape(1, cin)
    return pl.pallas_call(
        functools.partial(_zmm_kernel, n_real, br, bn),
        grid=(nb, nk),
        in_specs=[pl.BlockSpec((br, cin), lambda i, k: (i, 0)),
                  pl.BlockSpec((cin, cout), lambda i, k: (k, 0)),
                  pl.BlockSpec((1, cin), lambda i, k: (0, 0)),
                  pl.BlockSpec((1, cin), lambda i, k: (0, 0))],
        out_specs=pl.BlockSpec((br, cout), lambda i, k: (k * nb + i, 0)),
        out_shape=jax.ShapeDtypeStruct((nk * npad, cout), jnp.float32),
    )(xp, wf, s, b)


def _up_kernel(n_real, br, u_ref, wu_ref, kx_ref, skip_ref, s_ref, b_ref,
               s4_ref, b4_ref, o_ref):
    pid = pl.program_id(0)
    u = jnp.maximum(u_ref[...] * s4_ref[...] + b4_ref[...], 0.0)
    p = jnp.dot(u, wu_ref[...], preferred_element_type=jnp.float32)
    kx = kx_ref[...]
    up = jnp.zeros((br, 32), jnp.float32)
    for k in range(8):
        sel = (kx == k).astype(jnp.float32)
        up = up + sel * p[:, k * 32:(k + 1) * 32]
    cat = jnp.concatenate([skip_ref[...], up], axis=1)
    y = jnp.maximum(cat * s_ref[...] + b_ref[...], 0.0)
    rows = jax.lax.broadcasted_iota(jnp.int32, y.shape, 0) + pid * br
    o_ref[...] = jnp.where(rows < n_real, y, 0.0)


def _up_concat(u, wuf, kidx, skip, g5, b5, g4, b4, n_real, br=512):
    npad = u.shape[0]
    s = (g5 / np.sqrt(1.0 + EPS)).reshape(1, 64)
    b = b5.reshape(1, 64)
    s4 = (g4 / np.sqrt(1.0 + EPS)).reshape(1, 64)
    b4 = b4.reshape(1, 64)
    return pl.pallas_call(
        functools.partial(_up_kernel, n_real, br),
        grid=(npad // br,),
        in_specs=[pl.BlockSpec((br, 64), lambda i: (i, 0)),
                  pl.BlockSpec((64, 256), lambda i: (0, 0)),
                  pl.BlockSpec((br, 1), lambda i: (i, 0)),
                  pl.BlockSpec((br, 32), lambda i: (i, 0)),
                  pl.BlockSpec((1, 64), lambda i: (0, 0)),
                  pl.BlockSpec((1, 64), lambda i: (0, 0)),
                  pl.BlockSpec((1, 64), lambda i: (0, 0)),
                  pl.BlockSpec((1, 64), lambda i: (0, 0))],
        out_specs=pl.BlockSpec((br, 64), lambda i: (i, 0)),
        out_shape=jax.ShapeDtypeStruct((npad, 64), jnp.float32),
    )(u, wuf, kidx, skip, s, b, s4, b4)


def kernel(feat, pos, W1, Wd, W2, Wu, W3,
           g1, b1, g2, b2, g3, b3, g4, b4, g5, b5):
    n = feat.shape[0]
    chunk = ((n + 1 + NW - 1) // NW + IB - 1) // IB * IB
    npad = NW * chunk
    chh = chunk // 2
    nw2 = NW * 2

    i32 = jnp.int32
    posp = jnp.concatenate(
        [pos.astype(i32), jnp.full((npad - n, 3), -2, i32)], 0)
    posx = posp[:, 0] * 1
    posy = posp[:, 1] * 1
    posz = posp[:, 2] * 1
    featp = jnp.concatenate(
        [feat, jnp.zeros((npad - n, 32), feat.dtype)], 0)

    sd = jax.ShapeDtypeStruct
    vm = pltpu.VMEM

    t0, t1, poskey, k1key = _sc_call(
        functools.partial(_k1_body, chunk),
        (sd((T0SIZE,), i32), sd((T1SIZE,), i32),
         sd((npad,), i32), sd((npad,), i32)),
        [vm((chunk,), i32)] * 6 + [pltpu.SemaphoreType.DMA],
        posx, posy, posz)

    nbr0, dg, nbr1, kidx = _sc_call(
        functools.partial(_k2_body, chunk, n, npad),
        (sd((27 * npad,), i32), sd((8 * npad,), i32),
         sd((27 * npad,), i32), sd((npad,), i32)),
        [vm((chunk,), i32)] * 9 + [vm((chunk + 16,), i32)] * 4
        + [pltpu.SemaphoreType.DMA],
        posx, posy, posz, t0, t1, poskey, k1key, layout_passes=False)

    def compact_all(nbr0, dg, nbr1, caps):
        outs = []
        for cap in caps:
            outs += [sd((nw2 * cap,), i32), sd((nw2 * cap,), i32),
                     sd((nw2 * 16,), i32)]
        mcap = max(caps)
        res = _sc_call(
            functools.partial(_k4_body, chh, n, npad, caps),
            tuple(outs),
            [vm((27 * chh,), i32), vm((mcap,), i32), vm((mcap,), i32),
             vm((16,), i32)],
            nbr0, dg, nbr1, layout_passes=False)
        return [(tuple(res[3 * i:3 * i + 3]), caps[i]) for i in range(3)]

    def sconv(pairs, z, c):
        (dsts, srcs, cnts), cap = pairs
        return _sc_call(
            functools.partial(_k5_body, chh, c, cap),
            sd((npad, c), jnp.float32),
            [vm((cap + 16,), i32), vm((cap,), i32), vm((16,), i32),
             vm((256, c), jnp.float32), vm((chh + 8, c), jnp.float32),
             pltpu.SemaphoreType.DMA],
            dsts, srcs, cnts, z)

    def gather(idxs, f, nk, c):
        return _sc_call(
            functools.partial(_k3_body, chunk, npad, nk, c),
            sd((npad, nk * c), jnp.float32),
            [vm((chunk,), i32), vm((chunk, c), jnp.float32),
             pltpu.SemaphoreType.DMA],
            idxs, f)

    w1f = W1.reshape(27 * 32, 32)
    wdf = Wd.reshape(8 * 32, 64)
    w2f = W2.reshape(27 * 64, 64)
    wuf = Wu.transpose(1, 0, 2).reshape(64, 8 * 32)
    w3f = W3.reshape(27 * 64, 32)

    pairs0, pairsd, pairs1 = compact_all(nbr0, dg, nbr1, (4096, 2048, 8192))

    x_raw = sconv(pairs0, _zmm(featp, w1f, 27, n, g1, b1), 32)
    xd = sconv(pairsd, _zmm(x_raw, wdf, 8, n, g2, b2), 64)
    y_raw = sconv(pairs1, _zmm(xd, w2f, 27, n, g3, b3), 64)
    um = gather(nbr1[13 * npad:14 * npad], y_raw, 1, 64)
    x3p = _up_concat(um, wuf, kidx.reshape(npad, 1), x_raw,
                     g5, b5, g4, b4, n)
    out = sconv(pairs0, _zmm(x3p, w3f, 27, n), 32)
    return out[:n]
